# Initial kernel scaffold; baseline (speedup 1.0000x reference)
import jax
import jax.numpy as jnp
from jax.experimental import pallas as pl


def kernel(x_nd, r_ld):
    raise NotImplementedError("write your pallas kernel here")


# speedup vs baseline: 2.7180x; 2.7180x over previous
import functools
import math
import random

import jax
import jax.numpy as jnp
import numpy as np
from jax import lax
from jax.experimental import pallas as pl
from jax.experimental.pallas import tpu as pltpu
from jax.experimental.pallas import tpu_sc as plsc


def _algorithm_l_map(R, N):
    rng = random.Random(0)
    eps = 1e-06

    def u():
        return min(max(rng.random(), eps), 1.0 - eps)

    w_gen = 1.0
    current_index = R + 1
    cmap = {}
    while current_index <= N:
        candidate_idx = current_index - 1
        updated_idx = rng.randrange(R)
        cmap[updated_idx] = candidate_idx
        w_gen *= math.exp(math.log(u()) / R)
        w_gen = min(max(w_gen, 1e-06), 1.0 - 1e-06)
        current_index += math.floor(math.log(u()) / math.log(1.0 - w_gen)) + 1
    return cmap


@functools.lru_cache(maxsize=None)
def _src_indices(R, N):
    src = np.arange(R, dtype=np.int32)
    for slot, cand in _algorithm_l_map(R, N).items():
        src[slot] = cand
    return src


@functools.lru_cache(maxsize=None)
def _gather_kernel(N, R, D):
    info = plsc.get_sparse_core_info()
    NW = info.num_cores * info.num_subcores
    rows_per_w = R // NW
    CHUNK = 128
    NCHUNK = rows_per_w // CHUNK
    NBUF = 4

    mesh = plsc.VectorSubcoreMesh(core_axis_name="c", subcore_axis_name="s")

    @functools.partial(
        pl.kernel,
        mesh=mesh,
        compiler_params=pltpu.CompilerParams(use_tc_tiling_on_sc=False),
        out_type=jax.ShapeDtypeStruct((R, D), jnp.float32),
        scratch_types=[
            pltpu.VMEM((rows_per_w,), jnp.int32),
            pltpu.VMEM((NBUF, CHUNK, D), jnp.float32),
            pltpu.SemaphoreType.DMA((NBUF,)),
            pltpu.SemaphoreType.DMA((NBUF,)),
        ],
    )
    def k(x_hbm, idx_hbm, out_hbm, idx_v, rows_v, gsem, wsem):
        wid = lax.axis_index("s") * info.num_cores + lax.axis_index("c")
        base = wid * rows_per_w
        pltpu.sync_copy(idx_hbm.at[pl.ds(base, rows_per_w)], idx_v)

        def gather(j):
            b = j % NBUF
            return pltpu.async_copy(
                x_hbm.at[idx_v.at[pl.ds(j * CHUNK, CHUNK)]],
                rows_v.at[b],
                gsem.at[b],
            )

        def writeback(j):
            b = j % NBUF
            return pltpu.async_copy(
                rows_v.at[b],
                out_hbm.at[pl.ds(base + j * CHUNK, CHUNK)],
                wsem.at[b],
            )

        gcopies = {}
        wcopies = {}
        for j in range(min(NBUF, NCHUNK)):
            gcopies[j] = gather(j)
        for j in range(NCHUNK):
            gcopies[j].wait()
            wcopies[j] = writeback(j)
            nj = j + NBUF
            if nj < NCHUNK:
                wcopies[j].wait()
                gcopies[nj] = gather(nj)
        for j in range(max(0, NCHUNK - NBUF), NCHUNK):
            wcopies[j].wait()

    return k


def kernel(x_nd, r_ld):
    R, D = r_ld.shape
    N = x_nd.shape[0]
    src = jnp.asarray(_src_indices(R, N))
    return _gather_kernel(N, R, D)(x_nd, src)


# speedup vs baseline: 2.7277x; 1.0035x over previous
import functools
import math
import random

import jax
import jax.numpy as jnp
import numpy as np
from jax import lax
from jax.experimental import pallas as pl
from jax.experimental.pallas import tpu as pltpu
from jax.experimental.pallas import tpu_sc as plsc


def _algorithm_l_map(R, N):
    rng = random.Random(0)
    eps = 1e-06

    def u():
        return min(max(rng.random(), eps), 1.0 - eps)

    w_gen = 1.0
    current_index = R + 1
    cmap = {}
    while current_index <= N:
        candidate_idx = current_index - 1
        updated_idx = rng.randrange(R)
        cmap[updated_idx] = candidate_idx
        w_gen *= math.exp(math.log(u()) / R)
        w_gen = min(max(w_gen, 1e-06), 1.0 - 1e-06)
        current_index += math.floor(math.log(u()) / math.log(1.0 - w_gen)) + 1
    return cmap


@functools.lru_cache(maxsize=None)
def _src_indices(R, N):
    src = np.arange(R, dtype=np.int32)
    for slot, cand in _algorithm_l_map(R, N).items():
        src[slot] = cand
    return src


@functools.lru_cache(maxsize=None)
def _gather_kernel(N, R, D):
    info = plsc.get_sparse_core_info()
    NW = info.num_cores * info.num_subcores
    rows_per_w = R // NW
    CHUNK = 512
    NCHUNK = rows_per_w // CHUNK
    NBUF = 3

    mesh = plsc.VectorSubcoreMesh(core_axis_name="c", subcore_axis_name="s")

    @functools.partial(
        pl.kernel,
        mesh=mesh,
        compiler_params=pltpu.CompilerParams(use_tc_tiling_on_sc=False),
        out_type=jax.ShapeDtypeStruct((R, D), jnp.float32),
        scratch_types=[
            pltpu.VMEM((rows_per_w,), jnp.int32),
            pltpu.VMEM((NBUF, CHUNK, D), jnp.float32),
            pltpu.SemaphoreType.DMA((NBUF,)),
            pltpu.SemaphoreType.DMA((NBUF,)),
        ],
    )
    def k(x_hbm, idx_hbm, out_hbm, idx_v, rows_v, gsem, wsem):
        wid = lax.axis_index("s") * info.num_cores + lax.axis_index("c")
        base = wid * rows_per_w
        pltpu.sync_copy(idx_hbm.at[pl.ds(base, rows_per_w)], idx_v)

        def gather(j):
            b = j % NBUF
            return pltpu.async_copy(
                x_hbm.at[idx_v.at[pl.ds(j * CHUNK, CHUNK)]],
                rows_v.at[b],
                gsem.at[b],
            )

        def writeback(j):
            b = j % NBUF
            return pltpu.async_copy(
                rows_v.at[b],
                out_hbm.at[pl.ds(base + j * CHUNK, CHUNK)],
                wsem.at[b],
            )

        gcopies = {}
        wcopies = {}
        for j in range(min(NBUF, NCHUNK)):
            gcopies[j] = gather(j)
        for j in range(NCHUNK):
            gcopies[j].wait()
            wcopies[j] = writeback(j)
            nj = j + NBUF
            if nj < NCHUNK:
                wcopies[j].wait()
                gcopies[nj] = gather(nj)
        for j in range(max(0, NCHUNK - NBUF), NCHUNK):
            wcopies[j].wait()

    return k


def kernel(x_nd, r_ld):
    R, D = r_ld.shape
    N = x_nd.shape[0]
    src = jnp.asarray(_src_indices(R, N))
    return _gather_kernel(N, R, D)(x_nd, src)


# speedup vs baseline: 2.8507x; 1.0451x over previous
import functools
import math
import random

import jax
import jax.numpy as jnp
import numpy as np
from jax import lax
from jax.experimental import pallas as pl
from jax.experimental.pallas import tpu as pltpu
from jax.experimental.pallas import tpu_sc as plsc

_NW = 32
_CHUNK = 256
_NBUF = 3


def _algorithm_l_map(R, N):
    rng = random.Random(0)
    eps = 1e-06

    def u():
        return min(max(rng.random(), eps), 1.0 - eps)

    w_gen = 1.0
    current_index = R + 1
    cmap = {}
    while current_index <= N:
        candidate_idx = current_index - 1
        updated_idx = rng.randrange(R)
        cmap[updated_idx] = candidate_idx
        w_gen *= math.exp(math.log(u()) / R)
        w_gen = min(max(w_gen, 1e-06), 1.0 - 1e-06)
        current_index += math.floor(math.log(u()) / math.log(1.0 - w_gen)) + 1
    return cmap


@functools.lru_cache(maxsize=None)
def _fixup_table(R, N):
    src = np.arange(R, dtype=np.int64)
    for slot, cand in _algorithm_l_map(R, N).items():
        src[slot] = cand
    rows_per_w = R // _NW
    nchunk = rows_per_w // _CHUNK
    groups = [[[] for _ in range(nchunk)] for _ in range(_NW)]
    for i in range(R):
        if src[i] != i:
            w, r = divmod(i, rows_per_w)
            c, pos = divmod(r, _CHUNK)
            groups[w][c].append((pos << 17) | int(src[i]))
    maxfix = max(len(g) for row in groups for g in row)
    maxfix = (maxfix + 15) & ~15
    tab = np.empty((_NW, nchunk, maxfix), dtype=np.int32)
    for w in range(_NW):
        for c in range(nchunk):
            g = groups[w][c]
            pad = (0 << 17) | int(src[w * rows_per_w + c * _CHUNK])
            tab[w, c, :] = np.array(g + [pad] * (maxfix - len(g)), dtype=np.int32)
    return tab.reshape(-1), maxfix, nchunk


@functools.lru_cache(maxsize=None)
def _sampler_kernel(N, R, D, maxfix, nchunk):
    rows_per_w = R // _NW
    tab_per_w = nchunk * maxfix
    mesh = plsc.VectorSubcoreMesh(core_axis_name="c", subcore_axis_name="s")

    @functools.partial(
        pl.kernel,
        mesh=mesh,
        compiler_params=pltpu.CompilerParams(use_tc_tiling_on_sc=True),
        out_type=jax.ShapeDtypeStruct((R, D), jnp.float32),
        scratch_types=[
            pltpu.VMEM((tab_per_w,), jnp.int32),
            pltpu.VMEM((_NBUF, _CHUNK, D), jnp.float32),
            pltpu.SemaphoreType.DMA((_NBUF,)),
            pltpu.SemaphoreType.DMA((_NBUF,)),
            pltpu.SemaphoreType.DMA,
        ],
    )
    def k(x_hbm, tab_hbm, out_hbm, tab_v, rows_v, gsem, wsem, fsem):
        wid = lax.axis_index("s") * 2 + lax.axis_index("c")
        base = wid * rows_per_w
        pltpu.sync_copy(tab_hbm.at[pl.ds(wid * tab_per_w, tab_per_w)], tab_v)

        def bulk(j):
            b = j % _NBUF
            return pltpu.async_copy(
                x_hbm.at[pl.ds(base + j * _CHUNK, _CHUNK)], rows_v.at[b], gsem.at[b]
            )

        def writeback(j):
            b = j % _NBUF
            return pltpu.async_copy(
                rows_v.at[b], out_hbm.at[pl.ds(base + j * _CHUNK, _CHUNK)], wsem.at[b]
            )

        def fixups(j):
            b = j % _NBUF

            def body(g, _):
                vec = tab_v[pl.ds(j * maxfix + g * 16, 16)]
                for t in range(16):
                    v = vec[t]
                    pos = lax.shift_right_logical(v, 17)
                    srow = lax.bitwise_and(v, (1 << 17) - 1)
                    pltpu.make_async_copy(
                        x_hbm.at[pl.ds(srow, 1)], rows_v.at[b, pl.ds(pos, 1)], fsem
                    ).start()
                return 0

            lax.fori_loop(0, maxfix // 16, body, 0)
            pltpu.make_async_copy(
                x_hbm.at[pl.ds(0, maxfix)], rows_v.at[b, pl.ds(0, maxfix)], fsem
            ).wait()

        copies = {}
        wcopies = {}
        copies[0] = bulk(0)
        for j in range(nchunk):
            copies[j].wait()
            nj = j + 1
            if nj < nchunk:
                if nj >= _NBUF:
                    wcopies[nj - _NBUF].wait()
                copies[nj] = bulk(nj)
            fixups(j)
            wcopies[j] = writeback(j)
        for j in range(max(0, nchunk - _NBUF), nchunk):
            wcopies[j].wait()

    return k


def kernel(x_nd, r_ld):
    R, D = r_ld.shape
    N = x_nd.shape[0]
    tab, maxfix, nchunk = _fixup_table(R, N)
    return _sampler_kernel(N, R, D, maxfix, nchunk)(x_nd, jnp.asarray(tab))
